# baseline (speedup 1.0000x reference)
import functools
import math

import jax
import jax.numpy as jnp
from jax import lax
from jax.experimental import pallas as pl
from jax.experimental.pallas import tpu as pltpu
from jax.experimental.pallas import tpu_sc as plsc

_NLBL = 1000
_WPAD = 1024
_BLUR = 3
_DECAYS = tuple(math.exp(-float(d * d) / 2.0) for d in range(_BLUR + 1))

_B, _L = 4, 4096
_NC, _NS = 2, 16
_RPW = _NLBL // 8
_NV = _NLBL // 16
_TAIL = _NLBL - _NV * 16

_C6 = (-0.024825606615616704, 0.11790518317844773, -0.2723531579530551,
       0.4538562412335793, -0.7169868747326461, 1.442395482670534,
       5.065333099084653e-06)
_LN2 = 0.6931471805599453
_FMIN = -3.4e38


def _make_loss_kernel():
    mesh = plsc.VectorSubcoreMesh(core_axis_name="c", subcore_axis_name="s",
                                  num_cores=_NC, num_subcores=_NS)

    @functools.partial(
        pl.kernel,
        out_type=jax.ShapeDtypeStruct((_NC * 8,), jnp.float32),
        mesh=mesh,
        scratch_types=[
            pltpu.VMEM((_L,), jnp.int32),
            pltpu.VMEM((_WPAD,), jnp.float32),
            pltpu.VMEM((_WPAD,), jnp.float32),
            pltpu.VMEM((_WPAD,), jnp.float32),
            pltpu.VMEM((128,), jnp.float32),
            pltpu.VMEM((128,), jnp.float32),
            pltpu.VMEM((128,), jnp.float32),
            pltpu.VMEM((128,), jnp.float32),
            pltpu.VMEM((16,), jnp.float32),
            pltpu.VMEM((_NS * 8,), jnp.float32),
            pltpu.VMEM_SHARED((_NS * 8,), jnp.float32),
            pltpu.SemaphoreType.DMA,
            pltpu.SemaphoreType.DMA,
        ],
        compiler_params=pltpu.CompilerParams(needs_layout_passes=False),
    )
    def loss_kernel(pred_hbm, label_hbm, out_hbm, labels_v, wmap_v, buf0, buf1,
                    m_st, s_st, x0_st, w_st, part_v, accl, shared, sem0, sem1):
        c = lax.axis_index("c")
        s = lax.axis_index("s")
        bt = c * (_B // _NC) + (s >> 3)
        t0 = (s & 7) * _RPW

        def row_copy(i, buf, sem):
            off = pl.multiple_of((bt * _L + t0 + i) * _NLBL, 8)
            return pltpu.make_async_copy(
                pred_hbm.at[pl.ds(off, _NLBL)], buf.at[pl.ds(0, _NLBL)], sem)

        row_copy(0, buf0, sem0).start()
        row_copy(1, buf1, sem1).start()

        pltpu.sync_copy(label_hbm.at[pl.ds(pl.multiple_of(bt * _L, 8), _L)],
                        labels_v)

        zv = jnp.zeros((16,), jnp.float32)
        ov = jnp.full((16,), 1.0, jnp.float32)
        for k in range(_WPAD // 16):
            wmap_v[pl.ds(k * 16, 16)] = zv
        for k in range(8):
            sl = pl.ds(k * 16, 16)
            m_st[sl] = zv
            s_st[sl] = ov
            x0_st[sl] = zv
            w_st[sl] = zv

        for dist in range(_BLUR, -1, -1):
            for direction in (1, -1):
                off = direction * dist
                val = jnp.full((16,), _DECAYS[dist], jnp.float32)

                def body(j, carry, off=off, val=val):
                    lbl = labels_v[pl.ds(j * 16, 16)]
                    idx = jnp.clip(lbl + off, 0, _NLBL - 1)
                    plsc.store_scatter(wmap_v, [idx], val)
                    return carry

                lax.fori_loop(0, _L // 16, body, 0, unroll=4)
                if dist == 0:
                    break

        mask_tail = lax.iota(jnp.int32, 16) < _TAIL

        def process(buf, i):
            first = buf[pl.ds(0, 16)]
            m = first
            for k in range(1, _NV):
                m = jnp.maximum(m, buf[pl.ds(k * 16, 16)])
            last = buf[pl.ds(_NV * 16, 16)]
            m = jnp.maximum(m, jnp.where(mask_tail, last, _FMIN))
            m_sc = jnp.max(m)
            mv = jnp.full((16,), 1.0, jnp.float32) * m_sc
            acc = jnp.exp(first - mv)
            for k in range(1, _NV):
                acc = acc + jnp.exp(buf[pl.ds(k * 16, 16)] - mv)
            acc = acc + jnp.where(mask_tail, jnp.exp(last - mv), zv)
            lane = i & 15
            sl = pl.ds((i >> 4) * 16, 16)
            sel = lax.iota(jnp.int32, 16) == lane
            m_st[sl] = jnp.where(sel, m_sc, m_st[sl])
            s_st[sl] = jnp.where(sel, jnp.sum(acc), s_st[sl])
            x0_st[sl] = jnp.where(sel, first[0], x0_st[sl])
            wv = wmap_v[pl.ds(t0 + i, 16)]
            w_st[sl] = jnp.where(sel, wv[0], w_st[sl])

        def pair_body(j, carry):
            i0 = 2 * j
            row_copy(i0, buf0, sem0).wait()
            process(buf0, i0)
            row_copy(i0 + 2, buf0, sem0).start()
            row_copy(i0 + 1, buf1, sem1).wait()
            process(buf1, i0 + 1)
            row_copy(i0 + 3, buf1, sem1).start()
            return carry

        lax.fori_loop(0, _RPW // 2, pair_body, 0)
        row_copy(_RPW - 1, buf0, sem0).wait()
        process(buf0, _RPW - 1)
        row_copy(_RPW, buf1, sem1).wait()

        accv = jnp.zeros((16,), jnp.float32)
        for k in range(8):
            sl = pl.ds(k * 16, 16)
            sv = s_st[sl]
            bits = plsc.bitcast(sv, jnp.int32)
            ev = ((bits >> 23) - 127).astype(jnp.float32)
            mant = plsc.bitcast((bits & 0x007FFFFF) | 0x3F800000, jnp.float32)
            u = mant - 1.0
            p = jnp.full((16,), _C6[0], jnp.float32)
            for cf in _C6[1:]:
                p = p * u + cf
            lse = m_st[sl] + (ev + p) * _LN2
            accv = accv + w_st[sl] * (x0_st[sl] - lse)
        total = jnp.sum(accv)

        lane0 = lax.iota(jnp.int32, 16) == 0
        part_v[pl.ds(0, 16)] = jnp.where(lane0, total, 0.0)
        pltpu.sync_copy(part_v.at[pl.ds(0, 8)],
                        shared.at[pl.ds(pl.multiple_of(s * 8, 8), 8)])
        plsc.subcore_barrier()

        @pl.when(s == 0)
        def _merge():
            pltpu.sync_copy(shared, accl)
            tv = accl[pl.ds(0, 16)]
            for k in range(1, _NS // 2):
                tv = tv + accl[pl.ds(k * 16, 16)]
            tot = tv[0] + tv[8]
            part_v[pl.ds(0, 16)] = jnp.where(lane0, tot, 0.0)
            pltpu.sync_copy(part_v.at[pl.ds(0, 8)],
                            out_hbm.at[pl.ds(pl.multiple_of(c * 8, 8), 8)])

    return loss_kernel


def kernel(pred, label):
    B, L, C = pred.shape
    out = _make_loss_kernel()(pred.reshape(-1), label.reshape(-1))
    return -(out[0] + out[8]) / float(B * L)


# speedup vs baseline: 1.1382x; 1.1382x over previous
import functools
import math

import jax
import jax.numpy as jnp
from jax import lax
from jax.experimental import pallas as pl
from jax.experimental.pallas import tpu as pltpu
from jax.experimental.pallas import tpu_sc as plsc

_NLBL = 1000
_WPAD = 1024
_BLUR = 3
_DECAYS = tuple(math.exp(-float(d * d) / 2.0) for d in range(_BLUR + 1))

_B, _L = 4, 4096
_NC, _NS = 2, 16
_RPW = _NLBL // 8
_NV = _NLBL // 16
_TAIL = _NLBL - _NV * 16

_C6 = (-0.024825606615616704, 0.11790518317844773, -0.2723531579530551,
       0.4538562412335793, -0.7169868747326461, 1.442395482670534,
       5.065333099084653e-06)
_LN2 = 0.6931471805599453
_G = 25
_FMIN = -3.4e38


def _make_loss_kernel():
    mesh = plsc.VectorSubcoreMesh(core_axis_name="c", subcore_axis_name="s",
                                  num_cores=_NC, num_subcores=_NS)

    @functools.partial(
        pl.kernel,
        out_type=jax.ShapeDtypeStruct((_NC * 8,), jnp.float32),
        mesh=mesh,
        scratch_types=[
            pltpu.VMEM((_L,), jnp.int32),
            pltpu.VMEM((_WPAD,), jnp.float32),
            pltpu.VMEM((_G * _NLBL + 24,), jnp.float32),
            pltpu.VMEM((_G * _NLBL + 24,), jnp.float32),
            pltpu.VMEM((128,), jnp.float32),
            pltpu.VMEM((128,), jnp.float32),
            pltpu.VMEM((128,), jnp.float32),
            pltpu.VMEM((128,), jnp.float32),
            pltpu.VMEM((16,), jnp.float32),
            pltpu.VMEM((_NS * 8,), jnp.float32),
            pltpu.VMEM_SHARED((_NS * 8,), jnp.float32),
            pltpu.SemaphoreType.DMA,
            pltpu.SemaphoreType.DMA,
        ],
        compiler_params=pltpu.CompilerParams(needs_layout_passes=False),
    )
    def loss_kernel(pred_hbm, label_hbm, out_hbm, labels_v, wmap_v, buf0, buf1,
                    m_st, s_st, x0_st, w_st, part_v, accl, shared, sem0, sem1):
        c = lax.axis_index("c")
        s = lax.axis_index("s")
        bt = c * (_B // _NC) + (s >> 3)
        t0 = (s & 7) * _RPW

        def chunk_copy(ci, buf, sem):
            off = pl.multiple_of((bt * _L + t0 + ci * _G) * _NLBL, 8)
            return pltpu.make_async_copy(
                pred_hbm.at[pl.ds(off, _G * _NLBL)],
                buf.at[pl.ds(0, _G * _NLBL)], sem)

        chunk_copy(0, buf0, sem0).start()
        chunk_copy(1, buf1, sem1).start()

        pltpu.sync_copy(label_hbm.at[pl.ds(pl.multiple_of(bt * _L, 8), _L)],
                        labels_v)

        zv = jnp.zeros((16,), jnp.float32)
        ov = jnp.full((16,), 1.0, jnp.float32)
        for k in range(_WPAD // 16):
            wmap_v[pl.ds(k * 16, 16)] = zv
        for k in range(8):
            sl = pl.ds(k * 16, 16)
            m_st[sl] = zv
            s_st[sl] = ov
            x0_st[sl] = zv
            w_st[sl] = zv

        for dist in range(_BLUR, -1, -1):
            for direction in (1, -1):
                off = direction * dist
                val = jnp.full((16,), _DECAYS[dist], jnp.float32)

                def body(j, carry, off=off, val=val):
                    lbl = labels_v[pl.ds(j * 16, 16)]
                    idx = jnp.clip(lbl + off, 0, _NLBL - 1)
                    plsc.store_scatter(wmap_v, [idx], val)
                    return carry

                lax.fori_loop(0, _L // 16, body, 0, unroll=4)
                if dist == 0:
                    break

        mask_tail = lax.iota(jnp.int32, 16) < _TAIL

        def process(buf, base, i):
            first = buf[pl.ds(base, 16)]
            m = first
            for k in range(1, _NV):
                m = jnp.maximum(m, buf[pl.ds(base + k * 16, 16)])
            last = buf[pl.ds(base + _NV * 16, 16)]
            m = jnp.maximum(m, jnp.where(mask_tail, last, _FMIN))
            m_sc = jnp.max(m)
            mv = jnp.full((16,), 1.0, jnp.float32) * m_sc
            acc = jnp.exp(first - mv)
            for k in range(1, _NV):
                acc = acc + jnp.exp(buf[pl.ds(base + k * 16, 16)] - mv)
            acc = acc + jnp.where(mask_tail, jnp.exp(last - mv), zv)
            lane = i & 15
            sl = pl.ds((i >> 4) * 16, 16)
            sel = lax.iota(jnp.int32, 16) == lane
            m_st[sl] = jnp.where(sel, m_sc, m_st[sl])
            s_st[sl] = jnp.where(sel, jnp.sum(acc), s_st[sl])
            x0_st[sl] = jnp.where(sel, first[0], x0_st[sl])
            wv = wmap_v[pl.ds(t0 + i, 16)]
            w_st[sl] = jnp.where(sel, wv[0], w_st[sl])

        nchunks = _RPW // _G
        for ci in range(nchunks):
            buf, sem = (buf0, sem0) if ci % 2 == 0 else (buf1, sem1)
            chunk_copy(ci, buf, sem).wait()

            def row_body(rr, carry, buf=buf, ci=ci):
                process(buf, rr * _NLBL, ci * _G + rr)
                return carry

            lax.fori_loop(0, _G, row_body, 0)
            if ci + 2 < nchunks:
                chunk_copy(ci + 2, buf, sem).start()

        accv = jnp.zeros((16,), jnp.float32)
        for k in range(8):
            sl = pl.ds(k * 16, 16)
            sv = s_st[sl]
            bits = plsc.bitcast(sv, jnp.int32)
            ev = ((bits >> 23) - 127).astype(jnp.float32)
            mant = plsc.bitcast((bits & 0x007FFFFF) | 0x3F800000, jnp.float32)
            u = mant - 1.0
            p = jnp.full((16,), _C6[0], jnp.float32)
            for cf in _C6[1:]:
                p = p * u + cf
            lse = m_st[sl] + (ev + p) * _LN2
            accv = accv + w_st[sl] * (x0_st[sl] - lse)
        total = jnp.sum(accv)

        lane0 = lax.iota(jnp.int32, 16) == 0
        part_v[pl.ds(0, 16)] = jnp.where(lane0, total, 0.0)
        pltpu.sync_copy(part_v.at[pl.ds(0, 8)],
                        shared.at[pl.ds(pl.multiple_of(s * 8, 8), 8)])
        plsc.subcore_barrier()

        @pl.when(s == 0)
        def _merge():
            pltpu.sync_copy(shared, accl)
            tv = accl[pl.ds(0, 16)]
            for k in range(1, _NS // 2):
                tv = tv + accl[pl.ds(k * 16, 16)]
            tot = tv[0] + tv[8]
            part_v[pl.ds(0, 16)] = jnp.where(lane0, tot, 0.0)
            pltpu.sync_copy(part_v.at[pl.ds(0, 8)],
                            out_hbm.at[pl.ds(pl.multiple_of(c * 8, 8), 8)])

    return loss_kernel


def kernel(pred, label):
    B, L, C = pred.shape
    out = _make_loss_kernel()(pred.reshape(-1), label.reshape(-1))
    return -(out[0] + out[8]) / float(B * L)


# speedup vs baseline: 1.1600x; 1.0192x over previous
import functools
import math

import jax
import jax.numpy as jnp
from jax import lax
from jax.experimental import pallas as pl
from jax.experimental.pallas import tpu as pltpu
from jax.experimental.pallas import tpu_sc as plsc

_NLBL = 1000
_WPAD = 1024
_BLUR = 3
_DECAYS = tuple(math.exp(-float(d * d) / 2.0) for d in range(_BLUR + 1))

_B, _L = 4, 4096
_NC, _NS = 2, 16
_RPW = _NLBL // 8
_NV = _NLBL // 16
_TAIL = _NLBL - _NV * 16

_C6 = (-0.024825606615616704, 0.11790518317844773, -0.2723531579530551,
       0.4538562412335793, -0.7169868747326461, 1.442395482670534,
       5.065333099084653e-06)
_LN2 = 0.6931471805599453
_G = 25
_FMIN = -3.4e38


def _make_loss_kernel():
    mesh = plsc.VectorSubcoreMesh(core_axis_name="c", subcore_axis_name="s",
                                  num_cores=_NC, num_subcores=_NS)

    @functools.partial(
        pl.kernel,
        out_type=jax.ShapeDtypeStruct((_NC * 8,), jnp.float32),
        mesh=mesh,
        scratch_types=[
            pltpu.VMEM((_L,), jnp.int32),
            pltpu.VMEM((_WPAD,), jnp.float32),
            pltpu.VMEM((_G * _NLBL + 24,), jnp.float32),
            pltpu.VMEM((_G * _NLBL + 24,), jnp.float32),
            pltpu.VMEM((128,), jnp.float32),
            pltpu.VMEM((128,), jnp.float32),
            pltpu.VMEM((128,), jnp.float32),
            pltpu.VMEM((128,), jnp.float32),
            pltpu.VMEM((16,), jnp.float32),
            pltpu.VMEM((_NS * 8,), jnp.float32),
            pltpu.VMEM_SHARED((_NS * 8,), jnp.float32),
            pltpu.SemaphoreType.DMA,
            pltpu.SemaphoreType.DMA,
        ],
        compiler_params=pltpu.CompilerParams(needs_layout_passes=False),
    )
    def loss_kernel(pred_hbm, label_hbm, out_hbm, labels_v, wmap_v, buf0, buf1,
                    m_st, s_st, x0_st, w_st, part_v, accl, shared, sem0, sem1):
        c = lax.axis_index("c")
        s = lax.axis_index("s")
        bt = c * (_B // _NC) + (s >> 3)
        t0 = (s & 7) * _RPW

        def chunk_copy(ci, buf, sem):
            off = pl.multiple_of((bt * _L + t0 + ci * _G) * _NLBL, 8)
            return pltpu.make_async_copy(
                pred_hbm.at[pl.ds(off, _G * _NLBL)],
                buf.at[pl.ds(0, _G * _NLBL)], sem)

        chunk_copy(0, buf0, sem0).start()
        chunk_copy(1, buf1, sem1).start()

        pltpu.sync_copy(label_hbm.at[pl.ds(pl.multiple_of(bt * _L, 8), _L)],
                        labels_v)

        zv = jnp.zeros((16,), jnp.float32)
        ov = jnp.full((16,), 1.0, jnp.float32)
        for k in range(_WPAD // 16):
            wmap_v[pl.ds(k * 16, 16)] = zv
        for k in range(8):
            sl = pl.ds(k * 16, 16)
            m_st[sl] = zv
            s_st[sl] = ov
            x0_st[sl] = zv
            w_st[sl] = zv

        for dist in range(_BLUR, -1, -1):
            for direction in (1, -1):
                off = direction * dist
                val = jnp.full((16,), _DECAYS[dist], jnp.float32)

                def body(j, carry, off=off, val=val):
                    lbl = labels_v[pl.ds(j * 16, 16)]
                    idx = jnp.clip(lbl + off, 0, _NLBL - 1)
                    plsc.store_scatter(wmap_v, [idx], val)
                    return carry

                lax.fori_loop(0, _L // 16, body, 0, unroll=4)
                if dist == 0:
                    break

        mask_tail = lax.iota(jnp.int32, 16) < _TAIL

        def process(buf, base, i):
            NA = 8
            xs0 = [buf[pl.ds(base + a * 16, 16)] for a in range(NA)]
            ms = list(xs0)
            for k in range(NA, _NV):
                ms[k % NA] = jnp.maximum(ms[k % NA], buf[pl.ds(base + k * 16, 16)])
            last = buf[pl.ds(base + _NV * 16, 16)]
            ms[_NV % NA] = jnp.maximum(ms[_NV % NA],
                                       jnp.where(mask_tail, last, _FMIN))
            for stride in (4, 2, 1):
                for a in range(stride):
                    ms[a] = jnp.maximum(ms[a], ms[a + stride])
            m_sc = jnp.max(ms[0])
            mv = jnp.full((16,), 1.0, jnp.float32) * m_sc
            accs = [jnp.exp(xs0[a] - mv) for a in range(NA)]
            for k in range(NA, _NV):
                accs[k % NA] = accs[k % NA] + jnp.exp(
                    buf[pl.ds(base + k * 16, 16)] - mv)
            accs[_NV % NA] = accs[_NV % NA] + jnp.where(
                mask_tail, jnp.exp(last - mv), zv)
            for stride in (4, 2, 1):
                for a in range(stride):
                    accs[a] = accs[a] + accs[a + stride]
            lane = i & 15
            sl = pl.ds((i >> 4) * 16, 16)
            sel = lax.iota(jnp.int32, 16) == lane
            m_st[sl] = jnp.where(sel, m_sc, m_st[sl])
            s_st[sl] = jnp.where(sel, jnp.sum(accs[0]), s_st[sl])
            x0_st[sl] = jnp.where(sel, xs0[0][0], x0_st[sl])
            wv = wmap_v[pl.ds(t0 + i, 16)]
            w_st[sl] = jnp.where(sel, wv[0], w_st[sl])

        nchunks = _RPW // _G
        for ci in range(nchunks):
            buf, sem = (buf0, sem0) if ci % 2 == 0 else (buf1, sem1)
            chunk_copy(ci, buf, sem).wait()

            def row_body(rr, carry, buf=buf, ci=ci):
                process(buf, rr * _NLBL, ci * _G + rr)
                return carry

            lax.fori_loop(0, _G, row_body, 0)
            if ci + 2 < nchunks:
                chunk_copy(ci + 2, buf, sem).start()

        accv = jnp.zeros((16,), jnp.float32)
        for k in range(8):
            sl = pl.ds(k * 16, 16)
            sv = s_st[sl]
            bits = plsc.bitcast(sv, jnp.int32)
            ev = ((bits >> 23) - 127).astype(jnp.float32)
            mant = plsc.bitcast((bits & 0x007FFFFF) | 0x3F800000, jnp.float32)
            u = mant - 1.0
            p = jnp.full((16,), _C6[0], jnp.float32)
            for cf in _C6[1:]:
                p = p * u + cf
            lse = m_st[sl] + (ev + p) * _LN2
            accv = accv + w_st[sl] * (x0_st[sl] - lse)
        total = jnp.sum(accv)

        lane0 = lax.iota(jnp.int32, 16) == 0
        part_v[pl.ds(0, 16)] = jnp.where(lane0, total, 0.0)
        pltpu.sync_copy(part_v.at[pl.ds(0, 8)],
                        shared.at[pl.ds(pl.multiple_of(s * 8, 8), 8)])
        plsc.subcore_barrier()

        @pl.when(s == 0)
        def _merge():
            pltpu.sync_copy(shared, accl)
            tv = accl[pl.ds(0, 16)]
            for k in range(1, _NS // 2):
                tv = tv + accl[pl.ds(k * 16, 16)]
            tot = tv[0] + tv[8]
            part_v[pl.ds(0, 16)] = jnp.where(lane0, tot, 0.0)
            pltpu.sync_copy(part_v.at[pl.ds(0, 8)],
                            out_hbm.at[pl.ds(pl.multiple_of(c * 8, 8), 8)])

    return loss_kernel


def kernel(pred, label):
    B, L, C = pred.shape
    out = _make_loss_kernel()(pred.reshape(-1), label.reshape(-1))
    return -(out[0] + out[8]) / float(B * L)
